# baseline (speedup 1.0000x reference)
import jax
import jax.numpy as jnp
from jax.experimental import pallas as pl
from jax.experimental.pallas import tpu as pltpu

E = 64
D = 1024
FF = 1024
N = 2048
CAP = 40
CHUNK = 256
NCH = N // CHUNK


def _moe_body(x_ref, gw_ref, wg_ref, wu_ref, wo_ref, y_ref,
              eids_scr, pos_scr):
    e = pl.program_id(0)

    @pl.when(e == 0)
    def _prologue():
        y_ref[...] = jnp.zeros_like(y_ref)
        logits = jax.lax.dot_general(
            x_ref[...], gw_ref[...], (((1,), (1,)), ((), ())),
            preferred_element_type=jnp.float32)
        m = jnp.max(logits, axis=1, keepdims=True)
        eidx = jax.lax.broadcasted_iota(jnp.int32, (1, E), 1).astype(jnp.float32)
        eids = jnp.min(jnp.where(logits == m, eidx, jnp.float32(E)),
                       axis=1, keepdims=True)
        eids_scr[...] = eids

        tri_r = jax.lax.broadcasted_iota(jnp.int32, (CHUNK, CHUNK), 0)
        tri_c = jax.lax.broadcasted_iota(jnp.int32, (CHUNK, CHUNK), 1)
        tri = (tri_c <= tri_r).astype(jnp.float32)
        prefix = jnp.zeros((1, E), jnp.float32)
        for k in range(NCH):
            ek = eids_scr[k * CHUNK:(k + 1) * CHUNK, :]
            ok = (ek == eidx).astype(jnp.float32)
            pk = jax.lax.dot_general(tri, ok, (((1,), (0,)), ((), ())),
                                     preferred_element_type=jnp.float32)
            pk = pk + prefix
            prefix = pk[CHUNK - 1:CHUNK, :]
            pos_scr[k * CHUNK:(k + 1) * CHUNK, :] = (
                jnp.sum(ok * pk, axis=1, keepdims=True) - 1.0)

    ef = e.astype(jnp.float32)
    cidx = jax.lax.broadcasted_iota(jnp.int32, (1, CAP), 1).astype(jnp.float32)
    selT = ((eids_scr[...] == ef) & (pos_scr[...] == cidx)
            ).astype(jnp.float32)

    xg = jax.lax.dot_general(selT, x_ref[...], (((0,), (0,)), ((), ())),
                             precision=jax.lax.Precision.HIGHEST,
                             preferred_element_type=jnp.float32)
    g = jax.lax.dot_general(xg, wg_ref[0], (((1,), (1,)), ((), ())),
                            preferred_element_type=jnp.float32)
    u = jax.lax.dot_general(xg, wu_ref[0], (((1,), (1,)), ((), ())),
                            preferred_element_type=jnp.float32)
    h = (g * jax.nn.sigmoid(g)) * u
    part = jax.lax.dot_general(h, wo_ref[0], (((1,), (1,)), ((), ())),
                               preferred_element_type=jnp.float32)
    y_ref[...] += jax.lax.dot_general(selT, part, (((1,), (0,)), ((), ())),
                                      precision=jax.lax.Precision.HIGHEST,
                                      preferred_element_type=jnp.float32)


def kernel(x, gate_w, wi_gate, wi_up, wo):
    B, S, D_ = x.shape
    xf = x.reshape(N, D)

    y = pl.pallas_call(
        _moe_body,
        grid=(E,),
        in_specs=[
            pl.BlockSpec((N, D), lambda e: (0, 0)),
            pl.BlockSpec((E, D), lambda e: (0, 0)),
            pl.BlockSpec((1, FF, D), lambda e: (e, 0, 0)),
            pl.BlockSpec((1, FF, D), lambda e: (e, 0, 0)),
            pl.BlockSpec((1, D, FF), lambda e: (e, 0, 0)),
        ],
        out_specs=pl.BlockSpec((N, D), lambda e: (0, 0)),
        out_shape=jax.ShapeDtypeStruct((N, D), jnp.float32),
        scratch_shapes=[
            pltpu.VMEM((N, 1), jnp.float32),
            pltpu.VMEM((N, 1), jnp.float32),
        ],
    )(xf, gate_w, wi_gate, wi_up, wo)

    return y.reshape(B, S, D_)


# speedup vs baseline: 2.7436x; 2.7436x over previous
import jax
import jax.numpy as jnp
from jax.experimental import pallas as pl
from jax.experimental.pallas import tpu as pltpu

E = 64
D = 1024
FF = 1024
N = 2048
CAP = 40
CHUNK = 256
NCH = N // CHUNK


def _moe_body(x_ref, gw_ref, wg_ref, wu_ref, wo_ref, y_ref,
              eids_scr, pos_scr):
    e = pl.program_id(0)

    @pl.when(e == 0)
    def _prologue():
        y_ref[...] = jnp.zeros_like(y_ref)
        logits = jax.lax.dot_general(
            x_ref[...], gw_ref[...], (((1,), (1,)), ((), ())),
            preferred_element_type=jnp.float32)
        m = jnp.max(logits, axis=1, keepdims=True)
        eidx = jax.lax.broadcasted_iota(jnp.int32, (1, E), 1).astype(jnp.float32)
        eids = jnp.min(jnp.where(logits == m, eidx, jnp.float32(E)),
                       axis=1, keepdims=True)
        eids_scr[...] = eids

        tri_r = jax.lax.broadcasted_iota(jnp.int32, (CHUNK, CHUNK), 0)
        tri_c = jax.lax.broadcasted_iota(jnp.int32, (CHUNK, CHUNK), 1)
        tri = (tri_c <= tri_r).astype(jnp.float32)
        prefix = jnp.zeros((1, E), jnp.float32)
        for k in range(NCH):
            ek = eids_scr[k * CHUNK:(k + 1) * CHUNK, :]
            ok = (ek == eidx).astype(jnp.float32)
            pk = jax.lax.dot_general(tri, ok, (((1,), (0,)), ((), ())),
                                     preferred_element_type=jnp.float32)
            pk = pk + prefix
            prefix = pk[CHUNK - 1:CHUNK, :]
            pos_scr[k * CHUNK:(k + 1) * CHUNK, :] = (
                jnp.sum(ok * pk, axis=1, keepdims=True) - 1.0)

    ef = e.astype(jnp.float32)
    cidx = jax.lax.broadcasted_iota(jnp.int32, (1, CAP), 1).astype(jnp.float32)
    selT = ((eids_scr[...] == ef) & (pos_scr[...] == cidx)
            ).astype(jnp.float32)

    xg = jax.lax.dot_general(selT, x_ref[...], (((0,), (0,)), ((), ())),
                             preferred_element_type=jnp.float32)
    g = jax.lax.dot_general(xg, wg_ref[0], (((1,), (1,)), ((), ())),
                            preferred_element_type=jnp.float32)
    u = jax.lax.dot_general(xg, wu_ref[0], (((1,), (1,)), ((), ())),
                            preferred_element_type=jnp.float32)
    h = (g * jax.nn.sigmoid(g)) * u
    part = jax.lax.dot_general(h, wo_ref[0], (((1,), (1,)), ((), ())),
                               preferred_element_type=jnp.float32)
    y_ref[...] += jax.lax.dot_general(selT, part, (((1,), (0,)), ((), ())),
                                      preferred_element_type=jnp.float32)


def kernel(x, gate_w, wi_gate, wi_up, wo):
    B, S, D_ = x.shape
    xf = x.reshape(N, D)

    y = pl.pallas_call(
        _moe_body,
        grid=(E,),
        in_specs=[
            pl.BlockSpec((N, D), lambda e: (0, 0)),
            pl.BlockSpec((E, D), lambda e: (0, 0)),
            pl.BlockSpec((1, FF, D), lambda e: (e, 0, 0)),
            pl.BlockSpec((1, FF, D), lambda e: (e, 0, 0)),
            pl.BlockSpec((1, D, FF), lambda e: (e, 0, 0)),
        ],
        out_specs=pl.BlockSpec((N, D), lambda e: (0, 0)),
        out_shape=jax.ShapeDtypeStruct((N, D), jnp.float32),
        scratch_shapes=[
            pltpu.VMEM((N, 1), jnp.float32),
            pltpu.VMEM((N, 1), jnp.float32),
        ],
    )(xf, gate_w, wi_gate, wi_up, wo)

    return y.reshape(B, S, D_)
